# baseline (speedup 1.0000x reference)
import functools

import jax
import jax.numpy as jnp
from jax import lax
from jax.experimental import pallas as pl
from jax.experimental.pallas import tpu as pltpu
from jax.experimental.pallas import tpu_sc as plsc

B = 16384
D = 32
L = 16
NW = 32
BPW = B // NW


def _gmf_body(uid_hbm, iid_hbm, ut_hbm, it_hbm, wb_hbm, out_hbm,
              uidx, iidx, urows, irows, wv, outv, sem_u, sem_i):
    nc = 2
    wid = lax.axis_index("s") * nc + lax.axis_index("c")
    base = wid * BPW

    pltpu.sync_copy(uid_hbm.at[pl.ds(base, BPW)], uidx)
    pltpu.sync_copy(iid_hbm.at[pl.ds(base, BPW)], iidx)
    pltpu.sync_copy(wb_hbm, wv)

    cp_u = pltpu.async_copy(ut_hbm.at[uidx], urows, sem_u)
    cp_i = pltpu.async_copy(it_hbm.at[iidx], irows, sem_i)
    cp_u.wait()
    cp_i.wait()

    w_lo = wv[pl.ds(0, L)]
    w_hi = wv[pl.ds(L, L)]
    bias = wv[pl.ds(2 * L, L)][0]
    lane = lax.broadcasted_iota(jnp.int32, (L,), 0)

    def block(bi, carry):
        rb = pl.multiple_of(bi * L, L)
        acc = jnp.full((L,), bias, jnp.float32)
        for j in range(L):
            r = rb + j
            u0 = urows[r, pl.ds(0, L)]
            u1 = urows[r, pl.ds(L, L)]
            i0 = irows[r, pl.ds(0, L)]
            i1 = irows[r, pl.ds(L, L)]
            t = (u0 * i0) * w_lo + (u1 * i1) * w_hi
            e = [t[k] for k in range(L)]
            while len(e) > 1:
                e = [e[k] + e[k + 1] for k in range(0, len(e), 2)]
            acc = jnp.where(lane == j, acc + e[0], acc)
        outv[pl.ds(rb, L)] = 1.0 / (1.0 + jnp.exp(-acc))
        return carry

    lax.fori_loop(0, BPW // L, block, 0, unroll=False)

    pltpu.sync_copy(outv, out_hbm.at[pl.ds(base, BPW)])


@jax.jit
def _gmf(user_ids, item_ids, user_table, item_table, wb):
    mesh = plsc.VectorSubcoreMesh(core_axis_name="c", subcore_axis_name="s")
    f = functools.partial(
        pl.kernel,
        mesh=mesh,
        out_type=jax.ShapeDtypeStruct((B,), jnp.float32),
        scratch_types=[
            pltpu.VMEM((BPW,), jnp.int32),
            pltpu.VMEM((BPW,), jnp.int32),
            pltpu.VMEM((BPW, D), jnp.float32),
            pltpu.VMEM((BPW, D), jnp.float32),
            pltpu.VMEM((48,), jnp.float32),
            pltpu.VMEM((BPW,), jnp.float32),
            pltpu.SemaphoreType.DMA,
            pltpu.SemaphoreType.DMA,
        ],
        compiler_params=pltpu.CompilerParams(use_tc_tiling_on_sc=False),
    )(_gmf_body)
    return f(user_ids, item_ids, user_table, item_table, wb)


def kernel(user_ids, item_ids, user_table, item_table, fc_w, fc_b):
    wb = jnp.concatenate(
        [fc_w.reshape(-1), fc_b.reshape(-1),
         jnp.zeros((48 - D - 1,), jnp.float32)])
    out = _gmf(user_ids.astype(jnp.int32), item_ids.astype(jnp.int32),
               user_table, item_table, wb)
    return out.reshape(B, 1)
